# baseline (speedup 1.0000x reference)
import jax
import jax.numpy as jnp
from jax import lax
from jax.experimental import pallas as pl
from jax.experimental.pallas import tpu as pltpu
from jax.experimental.pallas import tpu_sc as plsc

BATCH = 1024
VOCAB = 100000
RB = 16
NI = BATCH // RB
K = int(0.7 * BATCH)


NBUF = 4


def _loss_body(x_hbm, t_ref, loss_ref, bufs, sems):
    i = pl.program_id(0)

    def copy(blk):
        return pltpu.make_async_copy(
            x_hbm.at[pl.ds(blk * RB, RB), :],
            bufs.at[lax.rem(blk, NBUF)],
            sems.at[lax.rem(blk, NBUF)])

    @pl.when(i == 0)
    def _prime():
        for b in range(NBUF - 1):
            copy(b).start()

    nxt = i + NBUF - 1

    @pl.when(nxt < NI)
    def _issue():
        copy(nxt).start()

    copy(i).wait()

    x = bufs[lax.rem(i, NBUF)]
    col = lax.broadcasted_iota(jnp.int32, (RB, VOCAB), 1)
    m = jnp.max(x, axis=1, keepdims=True)
    s = jnp.sum(jnp.exp(x - m), axis=1, keepdims=True)
    tv = jnp.sum(jnp.where(col == t_ref[...], x, 0.0), axis=1, keepdims=True)
    loss_ref[...] = m + jnp.log(s) - tv


def _per_sample_loss(x, tgt2d):
    return pl.pallas_call(
        _loss_body,
        grid=(NI,),
        in_specs=[
            pl.BlockSpec(memory_space=pl.ANY),
            pl.BlockSpec((RB, 1), lambda i: (i, 0)),
        ],
        out_specs=pl.BlockSpec((RB, 1), lambda i: (i, 0)),
        out_shape=jax.ShapeDtypeStruct((BATCH, 1), jnp.float32),
        scratch_shapes=[
            pltpu.VMEM((NBUF, RB, VOCAB), jnp.float32),
            pltpu.SemaphoreType.DMA((NBUF,)),
        ],
        compiler_params=pltpu.CompilerParams(
            dimension_semantics=("arbitrary",)),
    )(x, tgt2d)


_NVR = BATCH // 16
_I32_MIN = -2147483648
_I32_FLIP = 0x7FFFFFFF


def _topk_body(loss_hbm, out_hbm, loss_v, key_v, out_v):
    c = lax.axis_index("c")
    s = lax.axis_index("s")

    @pl.when(jnp.logical_and(c == 0, s == 0))
    def _work():
        pltpu.sync_copy(loss_hbm, loss_v)

        for i in range(_NVR):
            b = plsc.bitcast(loss_v[pl.ds(i * 16, 16)], jnp.int32)
            key_v[pl.ds(i * 16, 16)] = jnp.where(b < 0, b ^ _I32_FLIP, b)

        def count_ge(cand):
            acc = jnp.zeros((16,), jnp.int32)
            for i in range(_NVR):
                kv = key_v[pl.ds(i * 16, 16)]
                acc = acc + jnp.where(kv >= cand, 1, 0).astype(jnp.int32)
            return jnp.sum(acc)

        t0 = jnp.where(count_ge(jnp.int32(0)) >= K,
                       jnp.int32(0), jnp.int32(_I32_MIN))

        def bit_step(i, t):
            cand = t | lax.shift_left(jnp.int32(1), jnp.int32(30) - i)
            return jnp.where(count_ge(cand) >= K, cand, t)

        t = lax.fori_loop(0, 31, bit_step, t0)

        acc_sum = jnp.zeros((16,), jnp.float32)
        acc_cnt = jnp.zeros((16,), jnp.int32)
        for i in range(_NVR):
            kv = key_v[pl.ds(i * 16, 16)]
            xv = loss_v[pl.ds(i * 16, 16)]
            m = kv > t
            acc_sum = acc_sum + jnp.where(m, xv, 0.0)
            acc_cnt = acc_cnt + jnp.where(m, 1, 0).astype(jnp.int32)
        sum_gt = jnp.sum(acc_sum)
        cnt_gt = jnp.sum(acc_cnt)

        tbits = jnp.where(t < 0, t ^ _I32_FLIP, t)
        tval = plsc.bitcast(jnp.full((16,), tbits, jnp.int32), jnp.float32)
        mean_vec = (sum_gt + (K - cnt_gt).astype(jnp.float32) * tval) * (1.0 / K)
        out_v[...] = mean_vec
        pltpu.sync_copy(out_v, out_hbm)


def _topk_mean(loss1d):
    fn = pl.kernel(
        _topk_body,
        out_type=jax.ShapeDtypeStruct((16,), jnp.float32),
        mesh=plsc.VectorSubcoreMesh(core_axis_name="c", subcore_axis_name="s"),
        scratch_types=[
            pltpu.VMEM((BATCH,), jnp.float32),
            pltpu.VMEM((BATCH,), jnp.int32),
            pltpu.VMEM((16,), jnp.float32),
        ],
        compiler_params=pltpu.CompilerParams(needs_layout_passes=False),
    )
    return fn(loss1d)


def kernel(x, target):
    tgt2d = target.astype(jnp.int32).reshape(BATCH, 1)
    loss = _per_sample_loss(x, tgt2d)
    out16 = _topk_mean(loss.reshape(BATCH))
    return out16[0]
